# baseline (speedup 1.0000x reference)
import jax
import jax.numpy as jnp
from jax import lax
from jax.experimental import pallas as pl
from jax.experimental.pallas import tpu as pltpu
from jax.experimental.pallas import tpu_sc as plsc

N = 10000
D = 128
DH = D // 2
E = 320000
NC = 2
NS = 16
EPT = E // NS
CH = 80
NCHUNK = EPT // CH
ROWS_PT = 624
NBUF = 5
LOOK = 3


def _row_span(s):
    if s == NS - 1:
        return s * ROWS_PT, N - (NS - 1) * ROWS_PT
    return s * ROWS_PT, ROWS_PT


def _seg_sum_body(x_hbm, edge_hbm, out_hbm, xs_hbm,
                  src_v, dst_v, agg_sh, *bufs):
    rows = bufs[:NBUF]
    gsem = bufs[NBUF:2 * NBUF]
    ssem = bufs[2 * NBUF:]
    c = lax.axis_index("c")
    s = lax.axis_index("s")

    pltpu.sync_copy(edge_hbm.at[0, pl.ds(s * EPT, EPT)], src_v)
    pltpu.sync_copy(edge_hbm.at[1, pl.ds(s * EPT, EPT)], dst_v)

    for st in range(NS):
        @pl.when(s == st)
        def _():
            r0, nr = _row_span(st)
            half = x_hbm.at[pl.ds(r0, nr), pl.ds(c * DH, DH)]
            pltpu.sync_copy(half, xs_hbm.at[c].at[pl.ds(r0, nr)])
            pltpu.sync_copy(half, agg_sh.at[pl.ds(r0, nr)])
    plsc.subcore_barrier()

    tab = xs_hbm.at[c]
    for j in range(LOOK):
        pltpu.async_copy(tab.at[src_v.at[pl.ds(j * CH, CH)]], rows[j],
                         gsem[j])

    def _step(jj, carry):
        for b in range(NBUF):
            j = jj * NBUF + b
            nb = (b + LOOK) % NBUF

            def _wait_scatter():
                pltpu.make_async_copy(
                    rows[nb], agg_sh.at[dst_v.at[pl.ds(0, CH)]],
                    ssem[nb]).wait()

            if b + LOOK >= NBUF:
                _wait_scatter()
            else:
                pl.when(jj > 0)(_wait_scatter)

            @pl.when(j + LOOK < NCHUNK)
            def _():
                pltpu.async_copy(
                    tab.at[src_v.at[pl.ds((j + LOOK) * CH, CH)]], rows[nb],
                    gsem[nb])

            pltpu.make_async_copy(tab.at[src_v.at[pl.ds(j * CH, CH)]],
                                  rows[b], gsem[b]).wait()
            pltpu.async_copy(rows[b], agg_sh.at[dst_v.at[pl.ds(j * CH, CH)]],
                             ssem[b], add=True)
        return carry

    lax.fori_loop(0, NCHUNK // NBUF, _step, 0)
    for j in range(NCHUNK - (NBUF - LOOK), NCHUNK):
        pltpu.make_async_copy(rows[j % NBUF],
                              agg_sh.at[dst_v.at[pl.ds(0, CH)]],
                              ssem[j % NBUF]).wait()
    plsc.subcore_barrier()

    for st in range(NS):
        @pl.when(s == st)
        def _():
            r0, nr = _row_span(st)
            pltpu.sync_copy(agg_sh.at[pl.ds(r0, nr)],
                            out_hbm.at[pl.ds(r0, nr), pl.ds(c * DH, DH)])


def _seg_sum_sc(x, edge_index):
    mesh = plsc.VectorSubcoreMesh(core_axis_name="c", subcore_axis_name="s")
    h, _ = pl.kernel(
        _seg_sum_body,
        mesh=mesh,
        out_type=(jax.ShapeDtypeStruct((N, D), jnp.float32),
                  jax.ShapeDtypeStruct((NC, N, DH), jnp.float32)),
        scratch_types=(
            [pltpu.VMEM((EPT,), jnp.int32),
             pltpu.VMEM((EPT,), jnp.int32),
             pltpu.VMEM_SHARED((N, DH), jnp.float32)]
            + [pltpu.VMEM((CH, DH), jnp.float32)] * NBUF
            + [pltpu.SemaphoreType.DMA] * (2 * NBUF)
        ),
        compiler_params=pltpu.CompilerParams(use_tc_tiling_on_sc=False),
    )(x, edge_index)
    return h


def _bn_relu(t, g, be):
    m = jnp.mean(t, axis=0, keepdims=True)
    ms = jnp.mean(t * t, axis=0, keepdims=True)
    a = g * lax.rsqrt(ms - m * m + 1e-5)
    return jnp.maximum(t * a + (be - m * a), 0.0)


def _mlp_body(h_ref, w1_ref, b1_ref, g1_ref, be1_ref,
              w2_ref, b2_ref, g2_ref, be2_ref, o_ref):
    t = jnp.dot(h_ref[...], w1_ref[...],
                preferred_element_type=jnp.float32) + b1_ref[...]
    t = _bn_relu(t, g1_ref[...], be1_ref[...])
    u = jnp.dot(t, w2_ref[...], preferred_element_type=jnp.float32) + b2_ref[...]
    o_ref[...] = _bn_relu(u, g2_ref[...], be2_ref[...])


def _mlp_tc(h, W1, b1, g1, be1, W2, b2, g2, be2):
    return pl.pallas_call(
        _mlp_body,
        out_shape=jax.ShapeDtypeStruct((N, D), jnp.float32),
    )(h,
      W1, b1.reshape(1, -1), g1.reshape(1, -1), be1.reshape(1, -1),
      W2, b2.reshape(1, -1), g2.reshape(1, -1), be2.reshape(1, -1))


def kernel(x, edge_index, W1, b1, g1, be1, W2, b2, g2, be2):
    h = _seg_sum_sc(x, edge_index.astype(jnp.int32))
    return _mlp_tc(h, W1, b1, g1, be1, W2, b2, g2, be2)


# speedup vs baseline: 2.0297x; 2.0297x over previous
import jax
import jax.numpy as jnp
from jax import lax
from jax.experimental import pallas as pl
from jax.experimental.pallas import tpu as pltpu
from jax.experimental.pallas import tpu_sc as plsc

N = 10000
D = 128
DH = D // 2
E = 320000
NC = 2
NS = 16
EPT = E // NS
CH = 80
NCHUNK = EPT // CH
ROWS_PT = 624
NBUF = 5
LOOK = 3
ZROWS = 104


def _seg_sum_body(x2_hbm, src_hbm, dst_hbm, out_hbm,
                  src_v, dst_v, zbuf, agg_sh, *bufs):
    rows = bufs[:NBUF]
    gsem = bufs[NBUF:2 * NBUF]
    ssem = bufs[2 * NBUF:]
    c = lax.axis_index("c")
    s = lax.axis_index("s")

    pltpu.sync_copy(src_hbm.at[s], src_v)
    pltpu.sync_copy(dst_hbm.at[s], dst_v)

    def _xform(r, carry):
        for k in range(CH // 16):
            sl = pl.ds(k * 16, 16)
            src_v[r, sl] = src_v[r, sl] * 2 + c
        return carry
    lax.fori_loop(0, NCHUNK, _xform, 0)

    def _zrow(r, carry):
        for k in range(DH // 16):
            zbuf[r, pl.ds(k * 16, 16)] = jnp.zeros((16,), jnp.float32)
        return carry
    lax.fori_loop(0, ZROWS, _zrow, 0)
    for k in range(ROWS_PT // ZROWS):
        pltpu.sync_copy(zbuf, agg_sh.at[pl.ds(s * ROWS_PT + k * ZROWS, ZROWS)])

    @pl.when(s == NS - 1)
    def _():
        pltpu.sync_copy(zbuf.at[pl.ds(0, 16)],
                        agg_sh.at[pl.ds(NS * ROWS_PT, N - NS * ROWS_PT)])
    plsc.subcore_barrier()

    x_hbm = x2_hbm
    for j in range(LOOK):
        pltpu.async_copy(x_hbm.at[src_v.at[j]], rows[j], gsem[j])

    def _step(jj, carry):
        for b in range(NBUF):
            j = jj * NBUF + b
            nb = (b + LOOK) % NBUF

            def _wait_scatter():
                pltpu.make_async_copy(
                    rows[nb], agg_sh.at[dst_v.at[0]], ssem[nb]).wait()

            if b + LOOK >= NBUF:
                _wait_scatter()
            else:
                pl.when(jj > 0)(_wait_scatter)

            @pl.when(j + LOOK < NCHUNK)
            def _():
                pltpu.async_copy(x_hbm.at[src_v.at[j + LOOK]], rows[nb],
                                 gsem[nb])

            pltpu.make_async_copy(x_hbm.at[src_v.at[j]], rows[b],
                                  gsem[b]).wait()
            pltpu.async_copy(rows[b], agg_sh.at[dst_v.at[j]], ssem[b],
                             add=True)
        return carry

    lax.fori_loop(0, NCHUNK // NBUF, _step, 0)
    for j in range(NCHUNK - (NBUF - LOOK), NCHUNK):
        pltpu.make_async_copy(rows[j % NBUF], agg_sh.at[dst_v.at[0]],
                              ssem[j % NBUF]).wait()
    plsc.subcore_barrier()

    @pl.when(s < NS - 1)
    def _():
        pltpu.sync_copy(agg_sh.at[pl.ds(s * ROWS_PT, ROWS_PT)],
                        out_hbm.at[pl.ds(s * ROWS_PT, ROWS_PT),
                                   pl.ds(c * DH, DH)])

    @pl.when(s == NS - 1)
    def _():
        last = N - (NS - 1) * ROWS_PT
        pltpu.sync_copy(agg_sh.at[pl.ds((NS - 1) * ROWS_PT, last)],
                        out_hbm.at[pl.ds((NS - 1) * ROWS_PT, last),
                                   pl.ds(c * DH, DH)])


def _seg_sum_sc(x2, src_r, dst_r):
    mesh = plsc.VectorSubcoreMesh(core_axis_name="c", subcore_axis_name="s")
    return pl.kernel(
        _seg_sum_body,
        mesh=mesh,
        out_type=jax.ShapeDtypeStruct((N, D), jnp.float32),
        scratch_types=(
            [pltpu.VMEM((NCHUNK, CH), jnp.int32),
             pltpu.VMEM((NCHUNK, CH), jnp.int32),
             pltpu.VMEM((ZROWS, DH), jnp.float32),
             pltpu.VMEM_SHARED((N, DH), jnp.float32)]
            + [pltpu.VMEM((CH, DH), jnp.float32)] * NBUF
            + [pltpu.SemaphoreType.DMA] * (2 * NBUF)
        ),
        compiler_params=pltpu.CompilerParams(use_tc_tiling_on_sc=False),
    )(x2, src_r, dst_r)


def _bn_relu(t, g, be):
    m = jnp.mean(t, axis=0, keepdims=True)
    ms = jnp.mean(t * t, axis=0, keepdims=True)
    a = g * lax.rsqrt(ms - m * m + 1e-5)
    return jnp.maximum(t * a + (be - m * a), 0.0)


def _mlp_body(x_ref, a_ref, w1_ref, b1_ref, g1_ref, be1_ref,
              w2_ref, b2_ref, g2_ref, be2_ref, o_ref):
    t = jnp.dot(x_ref[...] + a_ref[...], w1_ref[...],
                preferred_element_type=jnp.float32) + b1_ref[...]
    t = _bn_relu(t, g1_ref[...], be1_ref[...])
    u = jnp.dot(t, w2_ref[...], preferred_element_type=jnp.float32) + b2_ref[...]
    o_ref[...] = _bn_relu(u, g2_ref[...], be2_ref[...])


def _mlp_tc(x, agg, W1, b1, g1, be1, W2, b2, g2, be2):
    return pl.pallas_call(
        _mlp_body,
        out_shape=jax.ShapeDtypeStruct((N, D), jnp.float32),
    )(x, agg,
      W1, b1.reshape(1, -1), g1.reshape(1, -1), be1.reshape(1, -1),
      W2, b2.reshape(1, -1), g2.reshape(1, -1), be2.reshape(1, -1))


def kernel(x, edge_index, W1, b1, g1, be1, W2, b2, g2, be2):
    ei = edge_index.astype(jnp.int32)
    src_r = ei[0].reshape(NS, NCHUNK, CH)
    dst_r = ei[1].reshape(NS, NCHUNK, CH)
    x2 = x.reshape(NC * N, DH)
    agg = _seg_sum_sc(x2, src_r, dst_r)
    return _mlp_tc(x, agg, W1, b1, g1, be1, W2, b2, g2, be2)
